# Initial kernel scaffold; baseline (speedup 1.0000x reference)
import jax
import jax.numpy as jnp
from jax.experimental import pallas as pl


def kernel(x, positive_edge_index, W1, b1, W2, b2):
    raise NotImplementedError("write your pallas kernel here")


# speedup vs baseline: 15.7737x; 15.7737x over previous
import functools

import jax
import jax.numpy as jnp
from jax import lax
from jax.experimental import pallas as pl
from jax.experimental.pallas import tpu as pltpu
from jax.experimental.pallas import tpu_sc as plsc

N = 10000
E = 320000
D = 128
NP = 10240

NC = 2
NS = 16
NW = NC * NS
EPT = E // NW
K = 80
UN = 5
CHO = EPT // (UN * K)
RPT = NP // NS
FB = 128
NF = RPT // FB

_mesh = plsc.VectorSubcoreMesh(core_axis_name="c", subcore_axis_name="s")


def _zero_rows(ref, nrows):
    def body(t, _):
        ref[t // 8, pl.ds((t % 8) * 16, 16)] = jnp.zeros((16,), jnp.float32)
        return 0
    lax.fori_loop(0, nrows * 8, body, 0)


@functools.partial(
    pl.kernel,
    out_type=jax.ShapeDtypeStruct((NC, NP, D), jnp.float32),
    mesh=_mesh,
    scratch_types=[
        pltpu.VMEM((UN, K), jnp.int32),
        pltpu.VMEM((K, D), jnp.float32),
        pltpu.VMEM((FB, D), jnp.float32),
        pltpu.VMEM_SHARED((NP, D), jnp.float32),
    ],
)
def _sc_hist(dst_hbm, out_hbm, dst_v, ones_v, bounce_v, acc_sh):
    c = lax.axis_index("c")
    s = lax.axis_index("s")
    wid = c * NS + s

    def ones_body(t, _):
        ones_v[t // 8, pl.ds((t % 8) * 16, 16)] = jnp.ones((16,), jnp.float32)
        return 0
    lax.fori_loop(0, K * 8, ones_body, 0)
    _zero_rows(bounce_v, FB)
    for t in range(NF):
        pltpu.sync_copy(bounce_v, acc_sh.at[pl.ds(s * RPT + t * FB, FB)])
    plsc.subcore_barrier()

    def chunk(i, _):
        pltpu.sync_copy(dst_hbm.at[wid].at[i], dst_v)
        for q in range(UN):
            pltpu.sync_copy(ones_v, acc_sh.at[dst_v.at[q]], add=True)
        return 0
    lax.fori_loop(0, CHO, chunk, 0)

    plsc.subcore_barrier()
    for t in range(NF):
        pltpu.sync_copy(acc_sh.at[pl.ds(s * RPT + t * FB, FB)], bounce_v)
        pltpu.sync_copy(bounce_v, out_hbm.at[c].at[pl.ds(s * RPT + t * FB, FB)])


@functools.partial(
    pl.kernel,
    out_type=jax.ShapeDtypeStruct((NC, NP, D), jnp.float32),
    mesh=_mesh,
    scratch_types=[
        pltpu.VMEM((UN, K), jnp.int32),
        pltpu.VMEM((UN, K), jnp.int32),
        pltpu.VMEM((K, D), jnp.float32),
        pltpu.VMEM((FB, D), jnp.float32),
        pltpu.VMEM_SHARED((NP, D), jnp.float32),
        pltpu.SemaphoreType.DMA,
    ],
)
def _sc_segsum(y_hbm, src_hbm, dst_hbm, out_hbm,
               src_v, dst_v, rows_v, bounce_v, acc_sh, sem):
    c = lax.axis_index("c")
    s = lax.axis_index("s")
    wid = c * NS + s

    _zero_rows(bounce_v, FB)
    for t in range(NF):
        pltpu.sync_copy(bounce_v, acc_sh.at[pl.ds(s * RPT + t * FB, FB)])
    plsc.subcore_barrier()

    def chunk(i, _):
        pltpu.sync_copy(src_hbm.at[wid].at[i], src_v)
        pltpu.sync_copy(dst_hbm.at[wid].at[i], dst_v)
        for q in range(UN):
            pltpu.async_copy(y_hbm.at[src_v.at[q]], rows_v, sem).wait()
            pltpu.sync_copy(rows_v, acc_sh.at[dst_v.at[q]], add=True)
        return 0
    lax.fori_loop(0, CHO, chunk, 0)

    plsc.subcore_barrier()
    for t in range(NF):
        pltpu.sync_copy(acc_sh.at[pl.ds(s * RPT + t * FB, FB)], bounce_v)
        pltpu.sync_copy(bounce_v, out_hbm.at[c].at[pl.ds(s * RPT + t * FB, FB)])


R = 1024
G = NP // R


def _tc_a_body(x_ref, w_ref, dinv_ref, invdeg_ref, y_ref, t_ref):
    xw = jnp.dot(x_ref[...], w_ref[...], preferred_element_type=jnp.float32)
    y_ref[...] = xw * dinv_ref[...]
    t_ref[...] = xw * invdeg_ref[...]


_tc_a = pl.pallas_call(
    _tc_a_body,
    grid=(G,),
    in_specs=[
        pl.BlockSpec((R, D), lambda i: (i, 0)),
        pl.BlockSpec((D, D), lambda i: (0, 0)),
        pl.BlockSpec((R, 1), lambda i: (i, 0)),
        pl.BlockSpec((R, 1), lambda i: (i, 0)),
    ],
    out_specs=[
        pl.BlockSpec((R, D), lambda i: (i, 0)),
        pl.BlockSpec((R, D), lambda i: (i, 0)),
    ],
    out_shape=[
        jax.ShapeDtypeStruct((NP, D), jnp.float32),
        jax.ShapeDtypeStruct((NP, D), jnp.float32),
    ],
)


def _tc_b_body(u_ref, t1_ref, dinv_ref, invdeg_ref, b1_ref, w2_ref,
               y_ref, t2_ref):
    dinv = dinv_ref[...]
    h = jnp.maximum(dinv * (u_ref[0] + u_ref[1]) + t1_ref[...] + b1_ref[...],
                    0.0)
    xw = jnp.dot(h, w2_ref[...], preferred_element_type=jnp.float32)
    y_ref[...] = xw * dinv
    t2_ref[...] = xw * invdeg_ref[...]


_tc_b = pl.pallas_call(
    _tc_b_body,
    grid=(G,),
    in_specs=[
        pl.BlockSpec((NC, R, D), lambda i: (0, i, 0)),
        pl.BlockSpec((R, D), lambda i: (i, 0)),
        pl.BlockSpec((R, 1), lambda i: (i, 0)),
        pl.BlockSpec((R, 1), lambda i: (i, 0)),
        pl.BlockSpec((1, D), lambda i: (0, 0)),
        pl.BlockSpec((D, D), lambda i: (0, 0)),
    ],
    out_specs=[
        pl.BlockSpec((R, D), lambda i: (i, 0)),
        pl.BlockSpec((R, D), lambda i: (i, 0)),
    ],
    out_shape=[
        jax.ShapeDtypeStruct((NP, D), jnp.float32),
        jax.ShapeDtypeStruct((NP, D), jnp.float32),
    ],
)


def _tc_c_body(v_ref, t2_ref, dinv_ref, b2_ref, o_ref):
    o_ref[...] = (dinv_ref[...] * (v_ref[0] + v_ref[1]) + t2_ref[...]
                  + b2_ref[...])


_tc_c = pl.pallas_call(
    _tc_c_body,
    grid=(G,),
    in_specs=[
        pl.BlockSpec((NC, R, D), lambda i: (0, i, 0)),
        pl.BlockSpec((R, D), lambda i: (i, 0)),
        pl.BlockSpec((R, 1), lambda i: (i, 0)),
        pl.BlockSpec((1, D), lambda i: (0, 0)),
    ],
    out_specs=pl.BlockSpec((R, D), lambda i: (i, 0)),
    out_shape=jax.ShapeDtypeStruct((NP, D), jnp.float32),
)


def kernel(x, positive_edge_index, W1, b1, W2, b2):
    src = positive_edge_index[0].astype(jnp.int32).reshape(NW, CHO, UN, K)
    dst = positive_edge_index[1].astype(jnp.int32).reshape(NW, CHO, UN, K)
    x_pad = jnp.zeros((NP, D), jnp.float32).at[:N].set(x)

    hist = _sc_hist(dst)
    deg = hist[0, :, 0] + hist[1, :, 0] + 1.0
    dinv = lax.rsqrt(deg).reshape(NP, 1)
    invdeg = (1.0 / deg).reshape(NP, 1)

    y1, t1 = _tc_a(x_pad, W1, dinv, invdeg)
    u = _sc_segsum(y1, src, dst)
    y2, t2 = _tc_b(u, t1, dinv, invdeg, b1.reshape(1, D), W2)
    v = _sc_segsum(y2, src, dst)
    out = _tc_c(v, t2, dinv, b2.reshape(1, D))
    return out[:N]


# speedup vs baseline: 20.4642x; 1.2974x over previous
import functools

import jax
import jax.numpy as jnp
from jax import lax
from jax.experimental import pallas as pl
from jax.experimental.pallas import tpu as pltpu
from jax.experimental.pallas import tpu_sc as plsc

N = 10000
E = 320000
D = 128
NP = 10240

NC = 2
NS = 16
NW = NC * NS
EPT = E // NW
K = 40
UN = 5
CHO = EPT // (UN * K)
RPT = NP // NS
FB = 128
NF = RPT // FB

_mesh = plsc.VectorSubcoreMesh(core_axis_name="c", subcore_axis_name="s")


def _zero_rows(ref, nrows):
    def body(t, _):
        ref[t // 8, pl.ds((t % 8) * 16, 16)] = jnp.zeros((16,), jnp.float32)
        return 0
    lax.fori_loop(0, nrows * 8, body, 0)


@functools.partial(
    pl.kernel,
    out_type=jax.ShapeDtypeStruct((NC, NP, D), jnp.float32),
    mesh=_mesh,
    scratch_types=[
        pltpu.VMEM((UN, K), jnp.int32),
        pltpu.VMEM((K, D), jnp.float32),
        pltpu.VMEM((FB, D), jnp.float32),
        pltpu.VMEM_SHARED((NP, D), jnp.float32),
    ],
)
def _sc_hist(dst_hbm, out_hbm, dst_v, ones_v, bounce_v, acc_sh):
    c = lax.axis_index("c")
    s = lax.axis_index("s")
    wid = c * NS + s

    def ones_body(t, _):
        ones_v[t // 8, pl.ds((t % 8) * 16, 16)] = jnp.ones((16,), jnp.float32)
        return 0
    lax.fori_loop(0, K * 8, ones_body, 0)
    _zero_rows(bounce_v, FB)
    for t in range(NF):
        pltpu.sync_copy(bounce_v, acc_sh.at[pl.ds(s * RPT + t * FB, FB)])
    plsc.subcore_barrier()

    def chunk(i, _):
        pltpu.sync_copy(dst_hbm.at[wid].at[i], dst_v)
        for q in range(UN):
            pltpu.sync_copy(ones_v, acc_sh.at[dst_v.at[q]], add=True)
        return 0
    lax.fori_loop(0, CHO, chunk, 0)

    plsc.subcore_barrier()
    for t in range(NF):
        pltpu.sync_copy(acc_sh.at[pl.ds(s * RPT + t * FB, FB)], bounce_v)
        pltpu.sync_copy(bounce_v, out_hbm.at[c].at[pl.ds(s * RPT + t * FB, FB)])


@functools.partial(
    pl.kernel,
    out_type=jax.ShapeDtypeStruct((NC, NP, D), jnp.float32),
    mesh=_mesh,
    scratch_types=[
        pltpu.VMEM((2, UN, K), jnp.int32),
        pltpu.VMEM((2, UN, K), jnp.int32),
        pltpu.VMEM((UN, K, D), jnp.float32),
        pltpu.VMEM((FB, D), jnp.float32),
        pltpu.VMEM_SHARED((NP, D), jnp.float32),
        pltpu.SemaphoreType.DMA((UN,)),
        pltpu.SemaphoreType.DMA((UN,)),
        pltpu.SemaphoreType.DMA((2,)),
    ],
)
def _sc_segsum(y_hbm, src_hbm, dst_hbm, out_hbm,
               src_v, dst_v, rows_v, bounce_v, acc_sh, gsem, ssem, isem):
    c = lax.axis_index("c")
    s = lax.axis_index("s")
    wid = c * NS + s

    _zero_rows(bounce_v, FB)
    for t in range(NF):
        pltpu.sync_copy(bounce_v, acc_sh.at[pl.ds(s * RPT + t * FB, FB)])
    pltpu.sync_copy(src_hbm.at[wid].at[0], src_v.at[0])
    pltpu.sync_copy(dst_hbm.at[wid].at[0], dst_v.at[0])
    plsc.subcore_barrier()

    def group(i, _):
        p = lax.rem(i, 2)

        @pl.when(i > 0)
        def _():
            for q in range(UN):
                pltpu.make_async_copy(
                    rows_v.at[q], acc_sh.at[dst_v.at[1 - p].at[q]],
                    ssem.at[q]).wait()

        @pl.when(i + 1 < CHO)
        def _():
            pltpu.async_copy(src_hbm.at[wid].at[i + 1], src_v.at[1 - p],
                             isem.at[0])
            pltpu.async_copy(dst_hbm.at[wid].at[i + 1], dst_v.at[1 - p],
                             isem.at[1])

        @pl.when(i > 0)
        def _():
            pltpu.make_async_copy(src_hbm.at[wid].at[i], src_v.at[p],
                                  isem.at[0]).wait()
            pltpu.make_async_copy(dst_hbm.at[wid].at[i], dst_v.at[p],
                                  isem.at[1]).wait()

        for q in range(UN):
            pltpu.async_copy(y_hbm.at[src_v.at[p].at[q]], rows_v.at[q],
                             gsem.at[q])
        for q in range(UN):
            pltpu.make_async_copy(y_hbm.at[src_v.at[p].at[q]], rows_v.at[q],
                                  gsem.at[q]).wait()
            pltpu.async_copy(rows_v.at[q], acc_sh.at[dst_v.at[p].at[q]],
                             ssem.at[q], add=True)
        return 0
    lax.fori_loop(0, CHO, group, 0)

    pfin = lax.rem(CHO - 1, 2)
    for q in range(UN):
        pltpu.make_async_copy(rows_v.at[q], acc_sh.at[dst_v.at[pfin].at[q]],
                              ssem.at[q]).wait()
    plsc.subcore_barrier()
    for t in range(NF):
        pltpu.sync_copy(acc_sh.at[pl.ds(s * RPT + t * FB, FB)], bounce_v)
        pltpu.sync_copy(bounce_v, out_hbm.at[c].at[pl.ds(s * RPT + t * FB, FB)])


R = 1024
G = NP // R


def _tc_a_body(x_ref, w_ref, dinv_ref, invdeg_ref, y_ref, t_ref):
    xw = jnp.dot(x_ref[...], w_ref[...], preferred_element_type=jnp.float32)
    y_ref[...] = xw * dinv_ref[...]
    t_ref[...] = xw * invdeg_ref[...]


_tc_a = pl.pallas_call(
    _tc_a_body,
    grid=(G,),
    in_specs=[
        pl.BlockSpec((R, D), lambda i: (i, 0)),
        pl.BlockSpec((D, D), lambda i: (0, 0)),
        pl.BlockSpec((R, 1), lambda i: (i, 0)),
        pl.BlockSpec((R, 1), lambda i: (i, 0)),
    ],
    out_specs=[
        pl.BlockSpec((R, D), lambda i: (i, 0)),
        pl.BlockSpec((R, D), lambda i: (i, 0)),
    ],
    out_shape=[
        jax.ShapeDtypeStruct((NP, D), jnp.float32),
        jax.ShapeDtypeStruct((NP, D), jnp.float32),
    ],
)


def _tc_b_body(u_ref, t1_ref, dinv_ref, invdeg_ref, b1_ref, w2_ref,
               y_ref, t2_ref):
    dinv = dinv_ref[...]
    h = jnp.maximum(dinv * (u_ref[0] + u_ref[1]) + t1_ref[...] + b1_ref[...],
                    0.0)
    xw = jnp.dot(h, w2_ref[...], preferred_element_type=jnp.float32)
    y_ref[...] = xw * dinv
    t2_ref[...] = xw * invdeg_ref[...]


_tc_b = pl.pallas_call(
    _tc_b_body,
    grid=(G,),
    in_specs=[
        pl.BlockSpec((NC, R, D), lambda i: (0, i, 0)),
        pl.BlockSpec((R, D), lambda i: (i, 0)),
        pl.BlockSpec((R, 1), lambda i: (i, 0)),
        pl.BlockSpec((R, 1), lambda i: (i, 0)),
        pl.BlockSpec((1, D), lambda i: (0, 0)),
        pl.BlockSpec((D, D), lambda i: (0, 0)),
    ],
    out_specs=[
        pl.BlockSpec((R, D), lambda i: (i, 0)),
        pl.BlockSpec((R, D), lambda i: (i, 0)),
    ],
    out_shape=[
        jax.ShapeDtypeStruct((NP, D), jnp.float32),
        jax.ShapeDtypeStruct((NP, D), jnp.float32),
    ],
)


def _tc_c_body(v_ref, t2_ref, dinv_ref, b2_ref, o_ref):
    o_ref[...] = (dinv_ref[...] * (v_ref[0] + v_ref[1]) + t2_ref[...]
                  + b2_ref[...])


_tc_c = pl.pallas_call(
    _tc_c_body,
    grid=(G,),
    in_specs=[
        pl.BlockSpec((NC, R, D), lambda i: (0, i, 0)),
        pl.BlockSpec((R, D), lambda i: (i, 0)),
        pl.BlockSpec((R, 1), lambda i: (i, 0)),
        pl.BlockSpec((1, D), lambda i: (0, 0)),
    ],
    out_specs=pl.BlockSpec((R, D), lambda i: (i, 0)),
    out_shape=jax.ShapeDtypeStruct((NP, D), jnp.float32),
)


def kernel(x, positive_edge_index, W1, b1, W2, b2):
    src = positive_edge_index[0].astype(jnp.int32).reshape(NW, CHO, UN, K)
    dst = positive_edge_index[1].astype(jnp.int32).reshape(NW, CHO, UN, K)
    x_pad = jnp.zeros((NP, D), jnp.float32).at[:N].set(x)

    hist = _sc_hist(dst)
    deg = hist[0, :, 0] + hist[1, :, 0] + 1.0
    dinv = lax.rsqrt(deg).reshape(NP, 1)
    invdeg = (1.0 / deg).reshape(NP, 1)

    y1, t1 = _tc_a(x_pad, W1, dinv, invdeg)
    u = _sc_segsum(y1, src, dst)
    y2, t2 = _tc_b(u, t1, dinv, invdeg, b1.reshape(1, D), W2)
    v = _sc_segsum(y2, src, dst)
    out = _tc_c(v, t2, dinv, b2.reshape(1, D))
    return out[:N]


# speedup vs baseline: 22.2679x; 1.0881x over previous
import functools

import jax
import jax.numpy as jnp
from jax import lax
from jax.experimental import pallas as pl
from jax.experimental.pallas import tpu as pltpu
from jax.experimental.pallas import tpu_sc as plsc

N = 10000
E = 320000
D = 128
NP = 10240

NC = 2
NS = 16
NW = NC * NS
EPT = E // NW
K = 40
UN = 5
CHO = EPT // (UN * K)
JP = CHO // 2
RPT = NP // NS
FB = 128
NF = RPT // FB

_mesh = plsc.VectorSubcoreMesh(core_axis_name="c", subcore_axis_name="s")


def _zero_rows(ref, nrows):
    def body(t, _):
        ref[t // 8, pl.ds((t % 8) * 16, 16)] = jnp.zeros((16,), jnp.float32)
        return 0
    lax.fori_loop(0, nrows * 8, body, 0)


def _flush(acc_sh, bounce_v, out_hbm, c, s):
    for t in range(NF):
        pltpu.sync_copy(acc_sh.at[pl.ds(s * RPT + t * FB, FB)], bounce_v)
        pltpu.sync_copy(bounce_v, out_hbm.at[c].at[pl.ds(s * RPT + t * FB, FB)])


@functools.partial(
    pl.kernel,
    out_type=jax.ShapeDtypeStruct((NC, NP, D), jnp.float32),
    mesh=_mesh,
    scratch_types=[
        pltpu.VMEM((2, UN, K), jnp.int32),
        pltpu.VMEM((K, D), jnp.float32),
        pltpu.VMEM((FB, D), jnp.float32),
        pltpu.VMEM_SHARED((NP, D), jnp.float32),
        pltpu.SemaphoreType.DMA((2,)),
        pltpu.SemaphoreType.DMA((2,)),
    ],
)
def _sc_hist(dst_hbm, out_hbm, dst_v, ones_v, bounce_v, acc_sh, ssem, isem):
    c = lax.axis_index("c")
    s = lax.axis_index("s")
    wid = c * NS + s

    def ones_body(t, _):
        ones_v[t // 8, pl.ds((t % 8) * 16, 16)] = jnp.ones((16,), jnp.float32)
        return 0
    lax.fori_loop(0, K * 8, ones_body, 0)
    _zero_rows(bounce_v, FB)
    for t in range(NF):
        pltpu.sync_copy(bounce_v, acc_sh.at[pl.ds(s * RPT + t * FB, FB)])
    pltpu.sync_copy(dst_hbm.at[wid].at[0], dst_v.at[0])
    plsc.subcore_barrier()

    def body(j, i, p):
        @pl.when(i > 0)
        def _():
            for q in range(UN):
                pltpu.make_async_copy(ones_v, acc_sh.at[dst_v.at[1 - p].at[q]],
                                      ssem.at[1 - p]).wait()
        @pl.when(i + 1 < CHO)
        def _():
            pltpu.async_copy(dst_hbm.at[wid].at[i + 1], dst_v.at[1 - p],
                             isem.at[1 - p])
        @pl.when(i > 0)
        def _():
            pltpu.make_async_copy(dst_hbm.at[wid].at[i], dst_v.at[p],
                                  isem.at[p]).wait()
        for q in range(UN):
            pltpu.async_copy(ones_v, acc_sh.at[dst_v.at[p].at[q]],
                             ssem.at[p], add=True)

    def pair(j, _):
        body(j, 2 * j, 0)
        body(j, 2 * j + 1, 1)
        return 0
    lax.fori_loop(0, JP, pair, 0)
    for q in range(UN):
        pltpu.make_async_copy(ones_v, acc_sh.at[dst_v.at[1].at[q]],
                              ssem.at[1]).wait()
    plsc.subcore_barrier()
    _flush(acc_sh, bounce_v, out_hbm, c, s)


@functools.partial(
    pl.kernel,
    out_type=jax.ShapeDtypeStruct((NC, NP, D), jnp.float32),
    mesh=_mesh,
    scratch_types=[
        pltpu.VMEM((2, UN, K), jnp.int32),
        pltpu.VMEM((2, UN, K), jnp.int32),
        pltpu.VMEM((UN, K, D), jnp.float32),
        pltpu.VMEM((FB, D), jnp.float32),
        pltpu.VMEM_SHARED((NP, D), jnp.float32),
        pltpu.SemaphoreType.DMA((UN,)),
        pltpu.SemaphoreType.DMA((2,)),
        pltpu.SemaphoreType.DMA((2,)),
    ],
)
def _sc_segsum(y_hbm, src_hbm, dst_hbm, out_hbm,
               src_v, dst_v, rows_v, bounce_v, acc_sh, gsem, ssem, isem):
    c = lax.axis_index("c")
    s = lax.axis_index("s")
    wid = c * NS + s

    _zero_rows(bounce_v, FB)
    for t in range(NF):
        pltpu.sync_copy(bounce_v, acc_sh.at[pl.ds(s * RPT + t * FB, FB)])
    pltpu.sync_copy(src_hbm.at[wid].at[0], src_v.at[0])
    pltpu.sync_copy(dst_hbm.at[wid].at[0], dst_v.at[0])
    plsc.subcore_barrier()

    def body(j, i, p):
        @pl.when(i > 0)
        def _():
            for q in range(UN):
                pltpu.make_async_copy(rows_v.at[q],
                                      acc_sh.at[dst_v.at[1 - p].at[q]],
                                      ssem.at[1 - p]).wait()
        @pl.when(i + 1 < CHO)
        def _():
            pltpu.async_copy(src_hbm.at[wid].at[i + 1], src_v.at[1 - p],
                             isem.at[1 - p])
            pltpu.async_copy(dst_hbm.at[wid].at[i + 1], dst_v.at[1 - p],
                             isem.at[1 - p])
        @pl.when(i > 0)
        def _():
            pltpu.make_async_copy(src_hbm.at[wid].at[i], src_v.at[p],
                                  isem.at[p]).wait()
            pltpu.make_async_copy(dst_hbm.at[wid].at[i], dst_v.at[p],
                                  isem.at[p]).wait()
        for q in range(UN):
            pltpu.async_copy(y_hbm.at[src_v.at[p].at[q]], rows_v.at[q],
                             gsem.at[q])
        for q in range(UN):
            pltpu.make_async_copy(y_hbm.at[src_v.at[p].at[q]], rows_v.at[q],
                                  gsem.at[q]).wait()
            pltpu.async_copy(rows_v.at[q], acc_sh.at[dst_v.at[p].at[q]],
                             ssem.at[p], add=True)

    def pair(j, _):
        body(j, 2 * j, 0)
        body(j, 2 * j + 1, 1)
        return 0
    lax.fori_loop(0, JP, pair, 0)
    for q in range(UN):
        pltpu.make_async_copy(rows_v.at[q], acc_sh.at[dst_v.at[1].at[q]],
                              ssem.at[1]).wait()
    plsc.subcore_barrier()
    _flush(acc_sh, bounce_v, out_hbm, c, s)


R = 1024
G = NP // R

_hist_spec = pl.BlockSpec((NC, R, D), lambda i: (0, i, 0))
_row_spec = pl.BlockSpec((R, D), lambda i: (i, 0))
_w_spec = pl.BlockSpec((D, D), lambda i: (0, 0))
_b_spec = pl.BlockSpec((1, D), lambda i: (0, 0))


def _deg_terms(hist_ref):
    deg = hist_ref[0, :, 0:1] + hist_ref[1, :, 0:1] + 1.0
    return lax.rsqrt(deg), 1.0 / deg


def _tc_a_body(x_ref, w_ref, hist_ref, y_ref, t_ref):
    xw = jnp.dot(x_ref[...], w_ref[...], preferred_element_type=jnp.float32)
    dinv, invdeg = _deg_terms(hist_ref)
    y_ref[...] = xw * dinv
    t_ref[...] = xw * invdeg


_tc_a = pl.pallas_call(
    _tc_a_body,
    grid=(G,),
    in_specs=[_row_spec, _w_spec, _hist_spec],
    out_specs=[_row_spec, _row_spec],
    out_shape=[
        jax.ShapeDtypeStruct((NP, D), jnp.float32),
        jax.ShapeDtypeStruct((NP, D), jnp.float32),
    ],
)


def _tc_b_body(u_ref, t1_ref, hist_ref, b1_ref, w2_ref, y_ref, t2_ref):
    dinv, invdeg = _deg_terms(hist_ref)
    h = jnp.maximum(dinv * (u_ref[0] + u_ref[1]) + t1_ref[...] + b1_ref[...],
                    0.0)
    xw = jnp.dot(h, w2_ref[...], preferred_element_type=jnp.float32)
    y_ref[...] = xw * dinv
    t2_ref[...] = xw * invdeg


_tc_b = pl.pallas_call(
    _tc_b_body,
    grid=(G,),
    in_specs=[
        pl.BlockSpec((NC, R, D), lambda i: (0, i, 0)),
        _row_spec, _hist_spec, _b_spec, _w_spec,
    ],
    out_specs=[_row_spec, _row_spec],
    out_shape=[
        jax.ShapeDtypeStruct((NP, D), jnp.float32),
        jax.ShapeDtypeStruct((NP, D), jnp.float32),
    ],
)


def _tc_c_body(v_ref, t2_ref, hist_ref, b2_ref, o_ref):
    dinv, _ = _deg_terms(hist_ref)
    o_ref[...] = dinv * (v_ref[0] + v_ref[1]) + t2_ref[...] + b2_ref[...]


_tc_c = pl.pallas_call(
    _tc_c_body,
    grid=(G,),
    in_specs=[
        pl.BlockSpec((NC, R, D), lambda i: (0, i, 0)),
        _row_spec, _hist_spec, _b_spec,
    ],
    out_specs=_row_spec,
    out_shape=jax.ShapeDtypeStruct((NP, D), jnp.float32),
)


def kernel(x, positive_edge_index, W1, b1, W2, b2):
    src = positive_edge_index[0].astype(jnp.int32).reshape(NW, CHO, UN, K)
    dst = positive_edge_index[1].astype(jnp.int32).reshape(NW, CHO, UN, K)
    x_pad = jnp.zeros((NP, D), jnp.float32).at[:N].set(x)

    hist = _sc_hist(dst)
    y1, t1 = _tc_a(x_pad, W1, hist)
    u = _sc_segsum(y1, src, dst)
    y2, t2 = _tc_b(u, t1, hist, b1.reshape(1, D), W2)
    v = _sc_segsum(y2, src, dst)
    out = _tc_c(v, t2, hist, b2.reshape(1, D))
    return out[:N]


# speedup vs baseline: 25.0536x; 1.1251x over previous
import functools

import jax
import jax.numpy as jnp
from jax import lax
from jax.experimental import pallas as pl
from jax.experimental.pallas import tpu as pltpu
from jax.experimental.pallas import tpu_sc as plsc

N = 10000
E = 320000
D = 128
NP = 10240

NC = 2
NS = 16
NW = NC * NS
EPT = E // NW
K = 40
UN = 5
CHO = EPT // (UN * K)
JP = CHO // 2
RPT = NP // NS
FB = 128
NF = RPT // FB

_mesh = plsc.VectorSubcoreMesh(core_axis_name="c", subcore_axis_name="s")


def _zero_rows(ref, nrows):
    def body(t, _):
        ref[t // 8, pl.ds((t % 8) * 16, 16)] = jnp.zeros((16,), jnp.float32)
        return 0
    lax.fori_loop(0, nrows * 8, body, 0)


def _flush(acc_sh, bounce_v, out_hbm, c, s):
    for t in range(NF):
        pltpu.sync_copy(acc_sh.at[pl.ds(s * RPT + t * FB, FB)], bounce_v)
        pltpu.sync_copy(bounce_v, out_hbm.at[c].at[pl.ds(s * RPT + t * FB, FB)])


HR = NP // D
HV = EPT // 16
KH = 80

@functools.partial(
    pl.kernel,
    out_type=jax.ShapeDtypeStruct((NC, HR, D), jnp.float32),
    mesh=_mesh,
    compiler_params=pltpu.CompilerParams(needs_layout_passes=False),
    scratch_types=[
        pltpu.VMEM((EPT // KH, KH), jnp.int32),
        pltpu.VMEM((HR, D), jnp.float32),
        pltpu.VMEM((HR, D), jnp.float32),
        pltpu.VMEM((HR,), jnp.int32),
        pltpu.VMEM_SHARED((HR, D), jnp.float32),
        pltpu.SemaphoreType.DMA,
    ],
)
def _sc_hist(dst_hbm, out_hbm, dst_v, hist_v, bounce_v, ident_v, acc_sh, sem):
    c = lax.axis_index("c")
    s = lax.axis_index("s")
    wid = c * NS + s

    pltpu.async_copy(dst_hbm.at[wid], dst_v, sem)
    _zero_rows(hist_v, HR)
    _zero_rows(bounce_v, HR)

    def ident_body(t, _):
        ident_v[pl.ds(t * 16, 16)] = lax.iota(jnp.int32, 16) + t * 16
        return 0
    lax.fori_loop(0, HR // 16, ident_body, 0)

    @pl.when(s == 0)
    def _():
        pltpu.sync_copy(bounce_v, acc_sh)
    pltpu.make_async_copy(dst_hbm.at[wid], dst_v, sem).wait()
    plsc.subcore_barrier()

    def count(t, _):
        vec = dst_v[t // 5, pl.ds((t % 5) * 16, 16)]
        cnt, last = plsc.scan_count(vec)
        plsc.addupdate_scatter(
            hist_v,
            [lax.shift_right_logical(vec, 7), lax.bitwise_and(vec, 127)],
            cnt.astype(jnp.float32), mask=last)
        return 0
    lax.fori_loop(0, HV, count, 0)

    pltpu.sync_copy(hist_v, acc_sh.at[ident_v], add=True)
    plsc.subcore_barrier()

    @pl.when(s == 0)
    def _():
        pltpu.sync_copy(acc_sh, bounce_v)
        pltpu.sync_copy(bounce_v, out_hbm.at[c])


@functools.partial(
    pl.kernel,
    out_type=jax.ShapeDtypeStruct((NC, NP, D), jnp.float32),
    mesh=_mesh,
    scratch_types=[
        pltpu.VMEM((2, UN, K), jnp.int32),
        pltpu.VMEM((2, UN, K), jnp.int32),
        pltpu.VMEM((UN, K, D), jnp.float32),
        pltpu.VMEM((FB, D), jnp.float32),
        pltpu.VMEM_SHARED((NP, D), jnp.float32),
        pltpu.SemaphoreType.DMA((UN,)),
        pltpu.SemaphoreType.DMA((2,)),
        pltpu.SemaphoreType.DMA((2,)),
    ],
)
def _sc_segsum(y_hbm, src_hbm, dst_hbm, out_hbm,
               src_v, dst_v, rows_v, bounce_v, acc_sh, gsem, ssem, isem):
    c = lax.axis_index("c")
    s = lax.axis_index("s")
    wid = c * NS + s

    _zero_rows(bounce_v, FB)
    for t in range(NF):
        pltpu.sync_copy(bounce_v, acc_sh.at[pl.ds(s * RPT + t * FB, FB)])
    pltpu.sync_copy(src_hbm.at[wid].at[0], src_v.at[0])
    pltpu.sync_copy(dst_hbm.at[wid].at[0], dst_v.at[0])
    plsc.subcore_barrier()

    def body(j, i, p):
        @pl.when(i > 0)
        def _():
            for q in range(UN):
                pltpu.make_async_copy(rows_v.at[q],
                                      acc_sh.at[dst_v.at[1 - p].at[q]],
                                      ssem.at[1 - p]).wait()
        @pl.when(i + 1 < CHO)
        def _():
            pltpu.async_copy(src_hbm.at[wid].at[i + 1], src_v.at[1 - p],
                             isem.at[1 - p])
            pltpu.async_copy(dst_hbm.at[wid].at[i + 1], dst_v.at[1 - p],
                             isem.at[1 - p])
        @pl.when(i > 0)
        def _():
            pltpu.make_async_copy(src_hbm.at[wid].at[i], src_v.at[p],
                                  isem.at[p]).wait()
            pltpu.make_async_copy(dst_hbm.at[wid].at[i], dst_v.at[p],
                                  isem.at[p]).wait()
        for q in range(UN):
            pltpu.async_copy(y_hbm.at[src_v.at[p].at[q]], rows_v.at[q],
                             gsem.at[q])
        for q in range(UN):
            pltpu.make_async_copy(y_hbm.at[src_v.at[p].at[q]], rows_v.at[q],
                                  gsem.at[q]).wait()
            pltpu.async_copy(rows_v.at[q], acc_sh.at[dst_v.at[p].at[q]],
                             ssem.at[p], add=True)

    def pair(j, _):
        body(j, 2 * j, 0)
        body(j, 2 * j + 1, 1)
        return 0
    lax.fori_loop(0, JP, pair, 0)
    for q in range(UN):
        pltpu.make_async_copy(rows_v.at[q], acc_sh.at[dst_v.at[1].at[q]],
                              ssem.at[1]).wait()
    plsc.subcore_barrier()
    _flush(acc_sh, bounce_v, out_hbm, c, s)


R = 1024
G = NP // R

_row_spec = pl.BlockSpec((R, D), lambda i: (i, 0))
_col_spec = pl.BlockSpec((R, 1), lambda i: (i, 0))
_w_spec = pl.BlockSpec((D, D), lambda i: (0, 0))
_b_spec = pl.BlockSpec((1, D), lambda i: (0, 0))


def _tc_a_body(x_ref, w_ref, dinv_ref, invdeg_ref, y_ref, t_ref):
    xw = jnp.dot(x_ref[...], w_ref[...], preferred_element_type=jnp.float32)
    y_ref[...] = xw * dinv_ref[...]
    t_ref[...] = xw * invdeg_ref[...]


_tc_a = pl.pallas_call(
    _tc_a_body,
    grid=(G,),
    in_specs=[_row_spec, _w_spec, _col_spec, _col_spec],
    out_specs=[_row_spec, _row_spec],
    out_shape=[
        jax.ShapeDtypeStruct((NP, D), jnp.float32),
        jax.ShapeDtypeStruct((NP, D), jnp.float32),
    ],
)


def _tc_b_body(u_ref, t1_ref, dinv_ref, invdeg_ref, b1_ref, w2_ref,
               y_ref, t2_ref):
    dinv = dinv_ref[...]
    h = jnp.maximum(dinv * (u_ref[0] + u_ref[1]) + t1_ref[...] + b1_ref[...],
                    0.0)
    xw = jnp.dot(h, w2_ref[...], preferred_element_type=jnp.float32)
    y_ref[...] = xw * dinv
    t2_ref[...] = xw * invdeg_ref[...]


_tc_b = pl.pallas_call(
    _tc_b_body,
    grid=(G,),
    in_specs=[
        pl.BlockSpec((NC, R, D), lambda i: (0, i, 0)),
        _row_spec, _col_spec, _col_spec, _b_spec, _w_spec,
    ],
    out_specs=[_row_spec, _row_spec],
    out_shape=[
        jax.ShapeDtypeStruct((NP, D), jnp.float32),
        jax.ShapeDtypeStruct((NP, D), jnp.float32),
    ],
)


def _tc_c_body(v_ref, t2_ref, dinv_ref, b2_ref, o_ref):
    o_ref[...] = (dinv_ref[...] * (v_ref[0] + v_ref[1]) + t2_ref[...]
                  + b2_ref[...])


_tc_c = pl.pallas_call(
    _tc_c_body,
    grid=(G,),
    in_specs=[
        pl.BlockSpec((NC, R, D), lambda i: (0, i, 0)),
        _row_spec, _col_spec, _b_spec,
    ],
    out_specs=_row_spec,
    out_shape=jax.ShapeDtypeStruct((NP, D), jnp.float32),
)


def kernel(x, positive_edge_index, W1, b1, W2, b2):
    src = positive_edge_index[0].astype(jnp.int32).reshape(NW, CHO, UN, K)
    dst = positive_edge_index[1].astype(jnp.int32).reshape(NW, CHO, UN, K)
    dst_h = positive_edge_index[1].astype(jnp.int32).reshape(NW, EPT // KH, KH)
    x_pad = jnp.zeros((NP, D), jnp.float32).at[:N].set(x)

    hist = _sc_hist(dst_h)
    deg = (hist[0] + hist[1]).reshape(NP) + 1.0
    dinv = lax.rsqrt(deg).reshape(NP, 1)
    invdeg = (1.0 / deg).reshape(NP, 1)

    y1, t1 = _tc_a(x_pad, W1, dinv, invdeg)
    u = _sc_segsum(y1, src, dst)
    y2, t2 = _tc_b(u, t1, dinv, invdeg, b1.reshape(1, D), W2)
    v = _sc_segsum(y2, src, dst)
    out = _tc_c(v, t2, dinv, b2.reshape(1, D))
    return out[:N]
